# baseline (speedup 1.0000x reference)
import functools

import jax
import jax.numpy as jnp
from jax.experimental import pallas as pl

N_TOKENS = 16384
HIDDEN = 2048
N_EXP = 16
BLK = 1024


def _gate_kernel(h_ref, w_ref, b_ref, tw_ref, ti_ref):
    h_blk = h_ref[...]
    w = w_ref[...]
    logits = jax.lax.dot_general(
        h_blk, w, (((1,), (1,)), ((), ())),
        preferred_element_type=jnp.float32)
    logits = logits + b_ref[...]

    idx1 = jnp.argmax(logits, axis=-1)
    l1 = jnp.max(logits, axis=-1)
    cols = jax.lax.broadcasted_iota(jnp.int32, logits.shape, 1)
    masked = jnp.where(cols == idx1[:, None], -jnp.inf, logits)
    idx2 = jnp.argmax(masked, axis=-1)
    l2 = jnp.max(masked, axis=-1)

    e2 = jnp.exp(l2 - l1)
    inv = 1.0 / (1.0 + e2)
    tw_ref[...] = jnp.stack([inv, e2 * inv], axis=-1)
    ti_ref[...] = jnp.stack([idx1, idx2], axis=-1).astype(jnp.int32)


@functools.partial(jax.jit, static_argnames=())
def kernel(h, weight, bias):
    n = h.shape[0]
    grid = (n // BLK,)
    tw, ti = pl.pallas_call(
        _gate_kernel,
        grid=grid,
        in_specs=[
            pl.BlockSpec((BLK, HIDDEN), lambda i: (i, 0)),
            pl.BlockSpec((N_EXP, HIDDEN), lambda i: (0, 0)),
            pl.BlockSpec((1, N_EXP), lambda i: (0, 0)),
        ],
        out_specs=[
            pl.BlockSpec((BLK, 2), lambda i: (i, 0)),
            pl.BlockSpec((BLK, 2), lambda i: (i, 0)),
        ],
        out_shape=[
            jax.ShapeDtypeStruct((n, 2), jnp.float32),
            jax.ShapeDtypeStruct((n, 2), jnp.int32),
        ],
    )(h, weight, bias.reshape(1, N_EXP))
    return (tw, ti)


# speedup vs baseline: 1.0105x; 1.0105x over previous
import functools

import jax
import jax.numpy as jnp
from jax.experimental import pallas as pl

N_TOKENS = 16384
HIDDEN = 2048
N_EXP = 16
BLK = 2048


def _gate_kernel(h_ref, w_ref, b_ref, tw_ref, ti_ref):
    h_blk = h_ref[...]
    w = w_ref[...]
    logits = jax.lax.dot_general(
        h_blk, w, (((1,), (1,)), ((), ())),
        preferred_element_type=jnp.float32)
    logits = logits + b_ref[...]

    idx1 = jnp.argmax(logits, axis=-1)
    l1 = jnp.max(logits, axis=-1)
    cols = jax.lax.broadcasted_iota(jnp.int32, logits.shape, 1)
    masked = jnp.where(cols == idx1[:, None], -jnp.inf, logits)
    idx2 = jnp.argmax(masked, axis=-1)
    l2 = jnp.max(masked, axis=-1)

    e2 = jnp.exp(l2 - l1)
    inv = 1.0 / (1.0 + e2)
    tw_ref[...] = jnp.stack([inv, e2 * inv], axis=-1)
    ti_ref[...] = jnp.stack([idx1, idx2], axis=-1).astype(jnp.int32)


@functools.partial(jax.jit, static_argnames=())
def kernel(h, weight, bias):
    n = h.shape[0]
    grid = (n // BLK,)
    tw, ti = pl.pallas_call(
        _gate_kernel,
        grid=grid,
        in_specs=[
            pl.BlockSpec((BLK, HIDDEN), lambda i: (i, 0)),
            pl.BlockSpec((N_EXP, HIDDEN), lambda i: (0, 0)),
            pl.BlockSpec((1, N_EXP), lambda i: (0, 0)),
        ],
        out_specs=[
            pl.BlockSpec((BLK, 2), lambda i: (i, 0)),
            pl.BlockSpec((BLK, 2), lambda i: (i, 0)),
        ],
        out_shape=[
            jax.ShapeDtypeStruct((n, 2), jnp.float32),
            jax.ShapeDtypeStruct((n, 2), jnp.int32),
        ],
    )(h, weight, bias.reshape(1, N_EXP))
    return (tw, ti)
